# baseline (speedup 1.0000x reference)
import functools

import jax
import jax.numpy as jnp
from jax import lax
from jax.experimental import pallas as pl
from jax.experimental.pallas import tpu as pltpu
from jax.experimental.pallas import tpu_sc as plsc

N = 10000
E = 320000
D = 128
NMASK = 1000

NC = 2
NS = 16
NW = NC * NS
CH = 64
NCHK = 5120
EP = NCHK * CH
CPW = NCHK // NW
EPW = CPW * CH
RPT = N // NS
ZCH = 125

f32 = jnp.float32
i32 = jnp.int32

_mesh = plsc.VectorSubcoreMesh(core_axis_name="c", subcore_axis_name="s")


@functools.partial(
    pl.kernel,
    out_type=jax.ShapeDtypeStruct((NC * N,), f32),
    mesh=_mesh,
    scratch_types=[
        pltpu.VMEM((CPW, CH), i32),
        pltpu.VMEM((EPW,), f32),
        pltpu.VMEM((1024,), f32),
        pltpu.VMEM_SHARED((N,), f32),
        pltpu.SemaphoreType.DMA,
        pltpu.SemaphoreType.DMA,
    ],
)
def _deg_kernel(dst2d_hbm, ew_hbm, out_hbm, idx_d, ewb, zb, sdeg, isem, ssem):
    c = lax.axis_index("c")
    s = lax.axis_index("s")
    wid = s * NC + c

    pltpu.async_copy(dst2d_hbm.at[pl.ds(wid * CPW, CPW)], idx_d, isem)
    pltpu.async_copy(ew_hbm.at[pl.ds(wid * EPW, EPW)], ewb, isem)

    def zb_body(i, carry):
        zb[pl.ds(i * 16, 16)] = jnp.zeros((16,), f32)
        return carry

    lax.fori_loop(0, 64, zb_body, 0)

    @pl.when(s < 10)
    def _():
        pltpu.sync_copy(zb.at[pl.ds(0, 1000)], sdeg.at[pl.ds(s * 1000, 1000)])

    plsc.subcore_barrier()
    pltpu.make_async_copy(dst2d_hbm.at[pl.ds(0, CPW)], idx_d, isem).wait()
    pltpu.make_async_copy(ew_hbm.at[pl.ds(0, EPW)], ewb, isem).wait()

    LAG = 6

    def chunk(i, carry):
        pltpu.async_copy(ewb.at[pl.ds(i * CH, CH)], sdeg.at[idx_d.at[i]],
                         ssem, add=True)

        @pl.when(i >= LAG)
        def _():
            pltpu.make_async_copy(ewb.at[pl.ds(0, CH)],
                                  sdeg.at[idx_d.at[0]], ssem).wait()

        return carry

    lax.fori_loop(0, CPW, chunk, 0)
    for _ in range(LAG):
        pltpu.make_async_copy(ewb.at[pl.ds(0, CH)],
                              sdeg.at[idx_d.at[0]], ssem).wait()

    plsc.subcore_barrier()

    @pl.when(s < 10)
    def _():
        pltpu.sync_copy(sdeg.at[pl.ds(s * 1000, 1000)], zb.at[pl.ds(0, 1000)])
        pltpu.sync_copy(zb.at[pl.ds(0, 1000)],
                        out_hbm.at[pl.ds(c * N + s * 1000, 1000)])


RING = 8
PF = 5


@functools.partial(
    pl.kernel,
    out_type=jax.ShapeDtypeStruct((NC, N, D), f32),
    mesh=_mesh,
    scratch_types=[
        pltpu.VMEM((RING, CH), i32),
        pltpu.VMEM((CPW, CH), i32),
        pltpu.VMEM((RING, CH), f32),
        pltpu.VMEM((3, CH, D), f32),
        pltpu.VMEM_SHARED((N, D), f32),
        pltpu.SemaphoreType.DMA,
        pltpu.SemaphoreType.DMA,
        pltpu.SemaphoreType.DMA,
        pltpu.SemaphoreType.DMA,
    ],
)
def _agg_kernel(src_hbm, dst2d_hbm, ew_hbm, g_hbm, out_hbm,
                idx_s, idx_d, ewb, rows3, acc, isem, psem, gsem, ssem):
    c = lax.axis_index("c")
    s = lax.axis_index("s")
    wid = s * NC + c

    pltpu.async_copy(dst2d_hbm.at[pl.ds(wid * CPW, CPW)], idx_d, isem)

    ebase = wid * EPW

    def issue_src(i):
        r = lax.rem(i, RING)
        pltpu.async_copy(src_hbm.at[pl.ds(ebase + i * CH, CH)],
                         idx_s.at[r], psem)
        pltpu.async_copy(ew_hbm.at[pl.ds(ebase + i * CH, CH)],
                         ewb.at[r], psem)

    def wait_src():
        pltpu.make_async_copy(src_hbm.at[pl.ds(0, CH)],
                              idx_s.at[0], psem).wait()
        pltpu.make_async_copy(ew_hbm.at[pl.ds(0, CH)],
                              ewb.at[0], psem).wait()

    for i in range(PF):
        issue_src(i)

    def zrow(i, carry):
        for k8 in range(8):
            rows3[0, i, pl.ds(k8 * 16, 16)] = jnp.zeros((16,), f32)
        return carry

    lax.fori_loop(0, CH, zrow, 0)

    rbase = s * RPT
    for k in range(9):
        pltpu.sync_copy(rows3.at[0, pl.ds(0, CH)],
                        acc.at[pl.ds(rbase + k * CH, CH)])
    pltpu.sync_copy(rows3.at[0, pl.ds(0, 49)],
                    acc.at[pl.ds(rbase + 9 * CH, 49)])
    plsc.subcore_barrier()

    pltpu.make_async_copy(dst2d_hbm.at[pl.ds(0, CPW)], idx_d, isem).wait()

    def issue_gather(i, b):
        pltpu.async_copy(g_hbm.at[idx_s.at[lax.rem(i, RING)]],
                         rows3.at[b], gsem)

    def wait_gather(b):
        pltpu.make_async_copy(g_hbm.at[idx_s.at[0]],
                              rows3.at[b], gsem).wait()

    def issue_scatter(i, b):
        pltpu.async_copy(rows3.at[b], acc.at[idx_d.at[i]], ssem, add=True)

    def wait_scatter(b):
        pltpu.make_async_copy(rows3.at[b], acc.at[idx_d.at[0]], ssem).wait()

    def scale(i, b):
        rr = lax.rem(i, RING)

        def grp(jv, carry):
            ew16 = ewb[rr, pl.ds(jv * 16, 16)]
            for lane in range(16):
                sc = ew16[lane]
                r = jv * 16 + lane
                for k8 in range(8):
                    sl = pl.ds(k8 * 16, 16)
                    rows3[b, r, sl] = rows3[b, r, sl] * sc
            return carry

        lax.fori_loop(0, CH // 16, grp, 0)

    for i in range(3):
        wait_src()
        issue_gather(i, i)
        issue_src(i + PF)
    wait_gather(0)
    scale(0, 0)
    issue_scatter(0, 0)
    wait_gather(1)
    scale(1, 1)
    issue_scatter(1, 1)

    def body(i, carry):
        b = lax.rem(i, 3)
        bp = lax.rem(i + 1, 3)
        wait_scatter(bp)

        @pl.when(i + 1 < CPW)
        def _():
            wait_src()
            issue_gather(i + 1, bp)

            @pl.when(i + 1 + PF < CPW)
            def _():
                issue_src(i + 1 + PF)

        wait_gather(b)
        scale(i, b)
        issue_scatter(i, b)
        return carry

    lax.fori_loop(2, CPW, body, 0)
    wait_scatter(0)
    wait_scatter(0)

    plsc.subcore_barrier()
    obase = s * 624
    for k in range(9):
        pltpu.sync_copy(acc.at[pl.ds(obase + k * CH, CH)],
                        rows3.at[0, pl.ds(0, CH)])
        pltpu.sync_copy(rows3.at[0, pl.ds(0, CH)],
                        out_hbm.at[c, pl.ds(obase + k * CH, CH)])
    pltpu.sync_copy(acc.at[pl.ds(obase + 9 * CH, 48)],
                    rows3.at[0, pl.ds(0, 48)])
    pltpu.sync_copy(rows3.at[0, pl.ds(0, 48)],
                    out_hbm.at[c, pl.ds(obase + 9 * CH, 48)])

    @pl.when(s == 0)
    def _():
        pltpu.sync_copy(acc.at[pl.ds(9984, 16)], rows3.at[0, pl.ds(0, 16)])
        pltpu.sync_copy(rows3.at[0, pl.ds(0, 16)],
                        out_hbm.at[c, pl.ds(9984, 16)])


MW = 25
MR = 40


@functools.partial(
    pl.kernel,
    out_type=(jax.ShapeDtypeStruct((NMASK, D), f32),
              jax.ShapeDtypeStruct((NMASK,), i32)),
    mesh=_mesh,
    scratch_types=[
        pltpu.VMEM((MR,), i32),
        pltpu.VMEM((MR, D), f32),
        pltpu.VMEM((MR,), i32),
        pltpu.SemaphoreType.DMA,
    ],
)
def _mask_kernel(outf_hbm, mask_hbm, y_hbm, om_hbm, ym_hbm,
                 midx, rowb, yb, sem):
    c = lax.axis_index("c")
    s = lax.axis_index("s")
    wid = s * NC + c

    @pl.when(wid < MW)
    def _():
        base = wid * MR
        pltpu.sync_copy(mask_hbm.at[pl.ds(base, MR)], midx)
        pltpu.async_copy(outf_hbm.at[midx], rowb, sem).wait()
        pltpu.sync_copy(rowb, om_hbm.at[pl.ds(base, MR)])
        pltpu.async_copy(y_hbm.at[midx], yb, sem).wait()
        pltpu.sync_copy(yb, ym_hbm.at[pl.ds(base, MR)])


BR = 2000


def _lin1_body(x_ref, w_ref, dp_ref, g_ref, dis_ref):
    deg = dp_ref[0] + dp_ref[1] + 1.0
    dis = jnp.where(deg > 0, lax.rsqrt(jnp.maximum(deg, 1e-12)), 0.0)
    h = jnp.dot(x_ref[...], w_ref[...], preferred_element_type=f32)
    g_ref[...] = h * dis
    dis_ref[...] = dis


def _lin1(x, W1, dp3):
    return pl.pallas_call(
        _lin1_body,
        grid=(N // BR,),
        in_specs=[
            pl.BlockSpec((BR, D), lambda i: (i, 0)),
            pl.BlockSpec((D, D), lambda i: (0, 0)),
            pl.BlockSpec((2, BR, 1), lambda i: (0, i, 0)),
        ],
        out_specs=[
            pl.BlockSpec((BR, D), lambda i: (i, 0)),
            pl.BlockSpec((BR, 1), lambda i: (i, 0)),
        ],
        out_shape=[
            jax.ShapeDtypeStruct((N, D), f32),
            jax.ShapeDtypeStruct((N, 1), f32),
        ],
    )(x, W1, dp3)


def _lin2_body(p_ref, g1_ref, dis_ref, b1_ref, w2_ref, g2_ref):
    t = dis_ref[...] * (p_ref[0] + p_ref[1] + g1_ref[...]) + b1_ref[...]
    z = jnp.maximum(t, 0.0)
    g2_ref[...] = jnp.dot(z, w2_ref[...],
                          preferred_element_type=f32) * dis_ref[...]


def _lin2(p, g1, dis, b1r, W2):
    return pl.pallas_call(
        _lin2_body,
        grid=(N // BR,),
        in_specs=[
            pl.BlockSpec((2, BR, D), lambda i: (0, i, 0)),
            pl.BlockSpec((BR, D), lambda i: (i, 0)),
            pl.BlockSpec((BR, 1), lambda i: (i, 0)),
            pl.BlockSpec((1, D), lambda i: (0, 0)),
            pl.BlockSpec((D, D), lambda i: (0, 0)),
        ],
        out_specs=pl.BlockSpec((BR, D), lambda i: (i, 0)),
        out_shape=jax.ShapeDtypeStruct((N, D), f32),
    )(p, g1, dis, b1r, W2)


def _combine_body(q_ref, g2_ref, dis_ref, b2_ref, o_ref):
    o_ref[...] = dis_ref[...] * (q_ref[0] + q_ref[1] + g2_ref[...]) \
        + b2_ref[...]


def _combine(q, g2, dis, b2r):
    return pl.pallas_call(
        _combine_body,
        grid=(N // BR,),
        in_specs=[
            pl.BlockSpec((2, BR, D), lambda i: (0, i, 0)),
            pl.BlockSpec((BR, D), lambda i: (i, 0)),
            pl.BlockSpec((BR, 1), lambda i: (i, 0)),
            pl.BlockSpec((1, D), lambda i: (0, 0)),
        ],
        out_specs=pl.BlockSpec((BR, D), lambda i: (i, 0)),
        out_shape=jax.ShapeDtypeStruct((N, D), f32),
    )(q, g2, dis, b2r)


def kernel(x, edge_index, edge_weight, mask_idx, y, W1, b1, W2, b2):
    pad = EP - E
    zpad_i = jnp.zeros((pad,), i32)
    src_idx = jnp.concatenate([edge_index[0], zpad_i])
    dst_idx = jnp.concatenate([edge_index[1], zpad_i])
    ew_p = jnp.concatenate([edge_weight, jnp.zeros((pad,), f32)])
    dst2d = dst_idx.reshape(NCHK, CH)

    dp = _deg_kernel(dst2d, ew_p)
    dp3 = dp.reshape(2, N, 1)
    g1, dis = _lin1(x, W1, dp3)
    p = _agg_kernel(src_idx, dst2d, ew_p, g1)
    g2 = _lin2(p, g1, dis, b1.reshape(1, D), W2)
    q = _agg_kernel(src_idx, dst2d, ew_p, g2)
    outf = _combine(q, g2, dis, b2.reshape(1, D))
    out_m, y_m = _mask_kernel(outf, mask_idx, y)
    return (out_m, y_m)
